# baseline (speedup 1.0000x reference)
import functools

import jax
import jax.numpy as jnp
from jax import lax
from jax.experimental import pallas as pl
from jax.experimental.pallas import tpu as pltpu
from jax.experimental.pallas import tpu_sc as plsc

_ROWS = 64
_DIM = 64
_STRIDE = 15625
_ROWS_PER_TILE = 16
_NUM_ACTIVE = _ROWS // _ROWS_PER_TILE

_mesh = plsc.VectorSubcoreMesh(core_axis_name="c", subcore_axis_name="s")


@functools.partial(
    pl.kernel,
    out_type=jax.ShapeDtypeStruct((_ROWS, _DIM), jnp.float32),
    mesh=_mesh,
    scratch_types=[
        pltpu.VMEM((_ROWS_PER_TILE, _DIM), jnp.float32),
        pltpu.SemaphoreType.DMA,
    ],
)
def _gather64(table_hbm, out_hbm, rows_v, sem):
    cid = lax.axis_index("c")
    sid = lax.axis_index("s")
    wid = sid * _mesh.num_cores + cid

    @pl.when(wid < _NUM_ACTIVE)
    def _():
        base = wid * _ROWS_PER_TILE
        copies = []
        for j in range(_ROWS_PER_TILE):
            row = (base + j) * _STRIDE
            copies.append(
                pltpu.async_copy(
                    table_hbm.at[pl.ds(row, 1)], rows_v.at[pl.ds(j, 1)], sem
                )
            )
        for c in copies:
            c.wait()
        pltpu.sync_copy(rows_v, out_hbm.at[pl.ds(base, _ROWS_PER_TILE)])


def kernel(table):
    return _gather64(table)
